# Initial kernel scaffold; baseline (speedup 1.0000x reference)
import jax
import jax.numpy as jnp
from jax.experimental import pallas as pl


def kernel(features, inds, Ws, bs):
    raise NotImplementedError("write your pallas kernel here")


# speedup vs baseline: 1.1032x; 1.1032x over previous
import functools

import jax
import jax.numpy as jnp
from jax import lax
from jax.experimental import pallas as pl
from jax.experimental.pallas import tpu as pltpu
from jax.experimental.pallas import tpu_sc as plsc

T = 4096
D = 768
E = 8
BT = 256
NTILES = -(-(T + E * (BT - 1)) // BT)
TPAD = NTILES * BT

NC = 2
NS = 16
NW = NC * NS
CHUNK = T // NW

@functools.cache
def _sc_kernels():
    mesh = plsc.VectorSubcoreMesh(core_axis_name="c", subcore_axis_name="s")

    @functools.partial(
        pl.kernel,
        mesh=mesh,
        out_type=jax.ShapeDtypeStruct((TPAD, D), jnp.float32),
        scratch_types=[
            pltpu.VMEM((CHUNK,), jnp.int32),
            pltpu.VMEM((CHUNK, D), jnp.float32),
            pltpu.SemaphoreType.DMA,
        ],
    )
    def sc_scatter(feat_hbm, pos_hbm, xpad_hbm, idx_v, rows_v, sem):
        wid = lax.axis_index("s") * NC + lax.axis_index("c")
        base = wid * CHUNK
        pltpu.sync_copy(feat_hbm.at[pl.ds(base, CHUNK)], rows_v)
        pltpu.sync_copy(pos_hbm.at[pl.ds(base, CHUNK)], idx_v)
        pltpu.async_copy(rows_v, xpad_hbm.at[idx_v], sem).wait()

    @functools.partial(
        pl.kernel,
        mesh=mesh,
        out_type=jax.ShapeDtypeStruct((T, D), jnp.float32),
        scratch_types=[
            pltpu.VMEM((CHUNK,), jnp.int32),
            pltpu.VMEM((CHUNK, D), jnp.float32),
            pltpu.SemaphoreType.DMA,
        ],
    )
    def sc_gather(y_hbm, pos_hbm, out_hbm, idx_v, rows_v, sem):
        wid = lax.axis_index("s") * NC + lax.axis_index("c")
        base = wid * CHUNK
        pltpu.sync_copy(pos_hbm.at[pl.ds(base, CHUNK)], idx_v)
        pltpu.async_copy(y_hbm.at[idx_v], rows_v, sem).wait()
        pltpu.sync_copy(rows_v, out_hbm.at[pl.ds(base, CHUNK)])

    return sc_scatter, sc_gather


def _mm_body(eid_ref, x_ref, w_ref, b_ref, o_ref):
    o_ref[...] = (
        jnp.dot(x_ref[...], w_ref[0], preferred_element_type=jnp.float32)
        + b_ref[0]
    )


_mm_call = pl.pallas_call(
    _mm_body,
    grid_spec=pltpu.PrefetchScalarGridSpec(
        num_scalar_prefetch=1,
        grid=(NTILES,),
        in_specs=[
            pl.BlockSpec((BT, D), lambda i, eid: (i, 0)),
            pl.BlockSpec((1, D, D), lambda i, eid: (eid[i], 0, 0)),
            pl.BlockSpec((1, 1, D), lambda i, eid: (eid[i], 0, 0)),
        ],
        out_specs=pl.BlockSpec((BT, D), lambda i, eid: (i, 0)),
    ),
    out_shape=jax.ShapeDtypeStruct((TPAD, D), jnp.float32),
)


def _route_meta(inds):
    inds32 = inds.astype(jnp.int32)
    oh = (inds32[:, None] == jnp.arange(E, dtype=jnp.int32)[None, :]).astype(
        jnp.int32
    )
    cum = jnp.cumsum(oh, axis=0)
    counts = cum[-1]
    rank = jnp.take_along_axis(cum - oh, inds32[:, None], axis=1)[:, 0]
    padded = ((counts + BT - 1) // BT) * BT
    poff = jnp.concatenate(
        [jnp.zeros((1,), jnp.int32), jnp.cumsum(padded)[:-1].astype(jnp.int32)]
    )
    pos = poff[inds32] + rank
    tile_starts = jnp.arange(NTILES, dtype=jnp.int32) * BT
    eid = jnp.clip(
        jnp.sum((poff[None, :] <= tile_starts[:, None]).astype(jnp.int32), axis=1)
        - 1,
        0,
        E - 1,
    ).astype(jnp.int32)
    return pos, eid


def kernel(features, inds, Ws, bs):
    sc_scatter, sc_gather = _sc_kernels()
    pos, eid = _route_meta(inds)
    x_pad = sc_scatter(features, pos)
    y_pad = _mm_call(eid, x_pad, Ws, bs.reshape(E, 1, D))
    out = sc_gather(y_pad, pos)
    return out


# speedup vs baseline: 1.1033x; 1.0001x over previous
import functools

import jax
import jax.numpy as jnp
from jax import lax
from jax.experimental import pallas as pl
from jax.experimental.pallas import tpu as pltpu
from jax.experimental.pallas import tpu_sc as plsc

T = 4096
D = 768
E = 8
BT = 256
NTILES = -(-(T + E * (BT - 1)) // BT)
TPAD = NTILES * BT

NC = 2
NS = 16
NW = NC * NS
CHUNK = T // NW

@functools.cache
def _sc_kernels():
    mesh = plsc.VectorSubcoreMesh(core_axis_name="c", subcore_axis_name="s")

    @functools.partial(
        pl.kernel,
        mesh=mesh,
        out_type=jax.ShapeDtypeStruct((TPAD, D), jnp.float32),
        scratch_types=[
            pltpu.VMEM((CHUNK,), jnp.int32),
            pltpu.VMEM((CHUNK, D), jnp.float32),
            pltpu.SemaphoreType.DMA,
        ],
    )
    def sc_scatter(feat_hbm, pos_hbm, xpad_hbm, idx_v, rows_v, sem):
        wid = lax.axis_index("s") * NC + lax.axis_index("c")
        base = wid * CHUNK
        pltpu.sync_copy(feat_hbm.at[pl.ds(base, CHUNK)], rows_v)
        pltpu.sync_copy(pos_hbm.at[pl.ds(base, CHUNK)], idx_v)
        pltpu.async_copy(rows_v, xpad_hbm.at[idx_v], sem).wait()

    @functools.partial(
        pl.kernel,
        mesh=mesh,
        out_type=jax.ShapeDtypeStruct((T, D), jnp.float32),
        scratch_types=[
            pltpu.VMEM((CHUNK,), jnp.int32),
            pltpu.VMEM((CHUNK, D), jnp.float32),
            pltpu.SemaphoreType.DMA,
        ],
    )
    def sc_gather(y_hbm, pos_hbm, out_hbm, idx_v, rows_v, sem):
        wid = lax.axis_index("s") * NC + lax.axis_index("c")
        base = wid * CHUNK
        pltpu.sync_copy(pos_hbm.at[pl.ds(base, CHUNK)], idx_v)
        pltpu.async_copy(y_hbm.at[idx_v], rows_v, sem).wait()
        pltpu.sync_copy(rows_v, out_hbm.at[pl.ds(base, CHUNK)])

    return sc_scatter, sc_gather


def _mm_body(eid_ref, x_ref, w_ref, b_ref, o_ref):
    o_ref[...] = (
        lax.dot_general(
            x_ref[...],
            w_ref[0],
            (((1,), (0,)), ((), ())),
            preferred_element_type=jnp.float32,
            precision=lax.Precision.DEFAULT,
        )
        + b_ref[0]
    )


_mm_call = pl.pallas_call(
    _mm_body,
    grid_spec=pltpu.PrefetchScalarGridSpec(
        num_scalar_prefetch=1,
        grid=(NTILES,),
        in_specs=[
            pl.BlockSpec((BT, D), lambda i, eid: (i, 0)),
            pl.BlockSpec((1, D, D), lambda i, eid: (eid[i], 0, 0)),
            pl.BlockSpec((1, 1, D), lambda i, eid: (eid[i], 0, 0)),
        ],
        out_specs=pl.BlockSpec((BT, D), lambda i, eid: (i, 0)),
    ),
    out_shape=jax.ShapeDtypeStruct((TPAD, D), jnp.float32),
)


def _route_meta(inds):
    inds32 = inds.astype(jnp.int32)
    oh = (inds32[:, None] == jnp.arange(E, dtype=jnp.int32)[None, :]).astype(
        jnp.int32
    )
    cum = jnp.cumsum(oh, axis=0)
    counts = cum[-1]
    rank = jnp.take_along_axis(cum - oh, inds32[:, None], axis=1)[:, 0]
    padded = ((counts + BT - 1) // BT) * BT
    poff = jnp.concatenate(
        [jnp.zeros((1,), jnp.int32), jnp.cumsum(padded)[:-1].astype(jnp.int32)]
    )
    pos = poff[inds32] + rank
    tile_starts = jnp.arange(NTILES, dtype=jnp.int32) * BT
    eid = jnp.clip(
        jnp.sum((poff[None, :] <= tile_starts[:, None]).astype(jnp.int32), axis=1)
        - 1,
        0,
        E - 1,
    ).astype(jnp.int32)
    return pos, eid


def kernel(features, inds, Ws, bs):
    sc_scatter, sc_gather = _sc_kernels()
    pos, eid = _route_meta(inds)
    x_pad = sc_scatter(features, pos)
    y_pad = _mm_call(eid, x_pad, Ws, bs.reshape(E, 1, D))
    out = sc_gather(y_pad, pos)
    return out
